# Initial kernel scaffold; baseline (speedup 1.0000x reference)
import functools

import jax
import jax.numpy as jnp
from jax import lax
from jax.experimental import pallas as pl
from jax.experimental.pallas import tpu as pltpu
from jax.experimental.pallas import tpu_sc as plsc

N = 50000
E = 800000
G = 256
F = 128
NQ = 12500
ACC_R = 12544
E_PAD = 802816
EROWS = E_PAD // 128
WROWS = EROWS // 32
NWIN = WROWS // 4
REGCAP = 25600
RING = 2048
NB = 50
R = 1000
EPS = 1e-5

_SC_PARAMS = dict(
    compiler_params=pltpu.CompilerParams(needs_layout_passes=False),
)


def _sc_mesh():
    return plsc.VectorSubcoreMesh(core_axis_name="c", subcore_axis_name="s",
                                  num_cores=2, num_subcores=16)


def _pick(vmem_ref, n_entries, target):
    tgt = jnp.full((16,), target, jnp.int32)
    acc16 = jnp.zeros((16,), jnp.int32)
    for g in range(n_entries // 16):
        vg = vmem_ref[pl.ds(g * 16, 16)]
        lane = lax.iota(jnp.int32, 16) + g * 16
        acc16 = jnp.maximum(acc16, jnp.where(lane == tgt, vg, 0))
    return jnp.max(acc16, axis=0)


def _deg_call(dst2):
    @functools.partial(
        pl.kernel,
        out_type=jax.ShapeDtypeStruct((2, 50176), jnp.float32),
        mesh=_sc_mesh(),
        **_SC_PARAMS,
        scratch_types=[
            pltpu.VMEM((4, 128), jnp.int32),
            pltpu.VMEM((128,), jnp.float32),
            pltpu.VMEM((3136,), jnp.float32),
            pltpu.VMEM_SHARED((50176,), jnp.float32),
        ],
    )
    def k(dst_ref, out, didx, ones_v, zbuf, acc):
        c = lax.axis_index("c")
        t = lax.axis_index("s")
        w = c * 16 + t
        one16 = jnp.full((16,), 1.0, jnp.float32)
        z16 = jnp.zeros((16,), jnp.float32)
        for i in range(8):
            ones_v[pl.ds(i * 16, 16)] = one16

        def zb(i, _):
            zbuf[pl.ds(i * 16, 16)] = z16
            return 0

        lax.fori_loop(0, 196, zb, 0)
        pltpu.sync_copy(zbuf, acc.at[pl.ds(t * 3136, 3136)])
        plsc.subcore_barrier()

        def win(wi, _):
            pltpu.sync_copy(dst_ref.at[pl.ds(w * WROWS + wi * 4, 4)], didx)
            for j in range(4):
                pltpu.sync_copy(ones_v, acc.at[didx.at[j]], add=True)
            return 0

        lax.fori_loop(0, NWIN, win, 0)
        plsc.subcore_barrier()
        pltpu.sync_copy(acc.at[pl.ds(t * 3136, 3136)], zbuf)
        pltpu.sync_copy(zbuf, out.at[c].at[pl.ds(t * 3136, 3136)])

    return k(dst2)


def _bin_call(src2, dst2, cc):
    @functools.partial(
        pl.kernel,
        out_type=[
            jax.ShapeDtypeStruct((4 * 32 * REGCAP,), jnp.int32),
            jax.ShapeDtypeStruct((4 * 32 * REGCAP,), jnp.int32),
            jax.ShapeDtypeStruct((256,), jnp.int32),
        ],
        mesh=_sc_mesh(),
        **_SC_PARAMS,
        scratch_types=[
            pltpu.VMEM((4, 128), jnp.int32),
            pltpu.VMEM((4, 128), jnp.int32),
            [pltpu.VMEM((RING,), jnp.int32) for _ in range(4)],
            [pltpu.VMEM((RING,), jnp.int32) for _ in range(4)],
            pltpu.VMEM((16,), jnp.int32),
            pltpu.VMEM((16,), jnp.int32),
        ],
    )
    def k(src_ref, dst_ref, cc_ref, bsrc, bdst, bcnt,
          swin, dwin, rs, rd, cbuf, ncv):
        c = lax.axis_index("c")
        t = lax.axis_index("s")
        w = c * 16 + t
        iota16 = lax.iota(jnp.int32, 16)
        pltpu.sync_copy(cc_ref, ncv)
        nwin = _pick(ncv, 16, 0)

        def win(wi, carry):
            cnts, fls = carry
            pltpu.sync_copy(src_ref.at[pl.ds(w * WROWS + wi * 4, 4)], swin)
            pltpu.sync_copy(dst_ref.at[pl.ds(w * WROWS + wi * 4, 4)], dwin)

            def grp(g, cs):
                c0, c1, c2, c3 = cs
                vd = dwin[g // 8, pl.ds((g % 8) * 16, 16)]
                vs = swin[g // 8, pl.ds((g % 8) * 16, 16)]
                b = ((vd >= NQ).astype(jnp.int32)
                     + (vd >= 2 * NQ).astype(jnp.int32)
                     + (vd >= 3 * NQ).astype(jnp.int32))
                dl = vd - b * NQ
                valid = vd < N
                outs = []
                for bi, cb in enumerate((c0, c1, c2, c3)):
                    msk = jnp.logical_and(b == bi, valid)
                    pos = plsc.cumsum(msk.astype(jnp.int32), mask=msk)
                    off = (cb + pos - 1) & (RING - 1)
                    plsc.store_scatter(rd[bi], [off], dl, mask=msk)
                    plsc.store_scatter(rs[bi], [off], vs, mask=msk)
                    npop = plsc.all_reduce_population_count(msk)
                    outs.append(cb + npop)
                return tuple(outs)

            cnts = lax.fori_loop(0, 32, grp, cnts)
            new_fls = []
            for bi in range(4):
                csc = jnp.max(cnts[bi], axis=0)
                fl = fls[bi]
                do = (csc - fl) >= 1024
                base = (bi * 32 + w) * REGCAP

                @pl.when(do)
                def _(bi=bi, fl=fl, base=base):
                    rb = pl.multiple_of(fl & (RING - 1), 1024)
                    ho = pl.multiple_of(base + fl, 512)
                    pltpu.sync_copy(rs[bi].at[pl.ds(rb, 1024)],
                                    bsrc.at[pl.ds(ho, 1024)])
                    pltpu.sync_copy(rd[bi].at[pl.ds(rb, 1024)],
                                    bdst.at[pl.ds(ho, 1024)])

                new_fls.append(jnp.where(do, fl + 1024, fl))
            return cnts, tuple(new_fls)

        z = jnp.zeros((16,), jnp.int32)
        zs = jnp.int32(0)
        (cnts, fls) = lax.fori_loop(
            0, nwin, win, ((z, z, z, z), (zs, zs, zs, zs)))

        cvec = jnp.zeros((16,), jnp.int32)
        for bi in range(4):
            csc = jnp.max(cnts[bi], axis=0)
            cpad = ((csc + 511) // 512) * 512
            cpad16 = jnp.full((16,), cpad, jnp.int32)
            for g in range(32):
                idx = csc + g * 16 + iota16
                msk = idx < cpad16
                off = idx & (RING - 1)
                plsc.store_scatter(rd[bi], [off],
                                   NQ + (idx & 31), mask=msk)
                plsc.store_scatter(rs[bi], [off],
                                   (idx * 37) & 32767, mask=msk)
            base = (bi * 32 + w) * REGCAP
            nrem = (cpad - fls[bi]) // 512

            def fin(r, _, bi=bi, base=base, fl=fls[bi]):
                off = fl + r * 512
                rb = pl.multiple_of(off & (RING - 1), 512)
                ho = pl.multiple_of(base + off, 512)
                pltpu.sync_copy(rs[bi].at[pl.ds(rb, 512)],
                                bsrc.at[pl.ds(ho, 512)])
                pltpu.sync_copy(rd[bi].at[pl.ds(rb, 512)],
                                bdst.at[pl.ds(ho, 512)])
                return 0

            lax.fori_loop(0, nrem, fin, 0)
            cvec = jnp.where(iota16 == bi, cpad, cvec)

        cbuf[pl.ds(0, 16)] = cvec
        pltpu.sync_copy(cbuf.at[pl.ds(0, 8)], bcnt.at[pl.ds(w * 8, 8)])

    return k(src2, dst2, cc)


def _agg_call(y, bsrc, bdst, bcnt):
    @functools.partial(
        pl.kernel,
        out_type=jax.ShapeDtypeStruct((N, F), jnp.float32),
        mesh=_sc_mesh(),
        **_SC_PARAMS,
        scratch_types=[
            pltpu.VMEM((512,), jnp.int32),
            pltpu.VMEM((4, 128), jnp.int32),
            pltpu.VMEM((512, F), jnp.float32),
            pltpu.VMEM((112, F), jnp.float32),
            pltpu.VMEM((256,), jnp.int32),
            pltpu.VMEM_SHARED((ACC_R, F), jnp.float32),
            pltpu.SemaphoreType.DMA,
        ],
    )
    def k(y_ref, bsrc_ref, bdst_ref, bcnt_ref, out,
          sidx, didx, rows, zbuf, cnv, acc, sem):
        c = lax.axis_index("c")
        t = lax.axis_index("s")
        z16 = jnp.zeros((16,), jnp.float32)

        def zb(i, _):
            zbuf[i // 8, pl.ds((i % 8) * 16, 16)] = z16
            return 0

        lax.fori_loop(0, 112 * 8, zb, 0)
        pltpu.sync_copy(bcnt_ref, cnv)

        for j in range(2):
            q = 2 * c + j
            for kk in range(7):
                pltpu.sync_copy(zbuf, acc.at[pl.ds(t * 784 + kk * 112, 112)])
            plsc.subcore_barrier()

            for rj in range(2):
                w = 2 * t + rj
                nw = _pick(cnv, 256, w * 8 + q) // 512
                base = (q * 32 + w) * REGCAP

                def win(wi, _, base=base):
                    off = pl.multiple_of(base + wi * 512, 512)
                    pltpu.sync_copy(bsrc_ref.at[pl.ds(off, 512)], sidx)
                    for jj in range(4):
                        pltpu.sync_copy(
                            bdst_ref.at[pl.ds(off + jj * 128, 128)],
                            didx.at[jj])
                    cps = [
                        pltpu.async_copy(
                            y_ref.at[sidx.at[pl.ds(jj * 128, 128)]],
                            rows.at[pl.ds(jj * 128, 128)], sem)
                        for jj in range(4)
                    ]
                    for cp in cps:
                        cp.wait()
                    for jj in range(4):
                        pltpu.sync_copy(rows.at[pl.ds(jj * 128, 128)],
                                        acc.at[didx.at[jj]], add=True)
                    return 0

                lax.fori_loop(0, nw, win, 0)

            plsc.subcore_barrier()
            @pl.when(t < 15)
            def _(q=q):
                pltpu.sync_copy(acc.at[pl.ds(t * 784, 784)],
                                out.at[pl.ds(q * NQ + t * 784, 784)])

            @pl.when(t == 15)
            def _(q=q):
                pltpu.sync_copy(acc.at[pl.ds(15 * 784, 740)],
                                out.at[pl.ds(q * NQ + 15 * 784, 740)])

            plsc.subcore_barrier()

    return k(y, bsrc, bdst, bcnt)


def _dis_block(deg_ref, i):
    d = (deg_ref[0, pl.ds(i * R, R)] + deg_ref[1, pl.ds(i * R, R)] + 1.0)
    return lax.rsqrt(d)[:, None]


def _tc1_call(x, w1, deg):
    def body(x_ref, w_ref, deg_ref, y_ref):
        i = pl.program_id(0)
        dis = _dis_block(deg_ref, i)
        xw = jnp.dot(x_ref[...], w_ref[...],
                     preferred_element_type=jnp.float32)
        y_ref[...] = xw * dis

    return pl.pallas_call(
        body,
        grid=(NB,),
        in_specs=[
            pl.BlockSpec((R, F), lambda i: (i, 0)),
            pl.BlockSpec((F, F), lambda i: (0, 0)),
            pl.BlockSpec((2, 50176), lambda i: (0, 0)),
        ],
        out_specs=pl.BlockSpec((R, F), lambda i: (i, 0)),
        out_shape=jax.ShapeDtypeStruct((N, F), jnp.float32),
    )(x, w1, deg)


def _stats_call(agg, y, deg, b):
    def body(agg_ref, y_ref, deg_ref, b_ref, pre_ref, st_ref):
        i = pl.program_id(0)
        dis = _dis_block(deg_ref, i)
        pre = (agg_ref[...] + y_ref[...]) * dis + b_ref[...]
        pre_ref[...] = pre

        @pl.when(i == 0)
        def _():
            st_ref[...] = jnp.zeros_like(st_ref)

        st_ref[0:1, :] = st_ref[0:1, :] + jnp.sum(pre, 0, keepdims=True)
        st_ref[1:2, :] = st_ref[1:2, :] + jnp.sum(pre * pre, 0, keepdims=True)

    return pl.pallas_call(
        body,
        grid=(NB,),
        in_specs=[
            pl.BlockSpec((R, F), lambda i: (i, 0)),
            pl.BlockSpec((R, F), lambda i: (i, 0)),
            pl.BlockSpec((2, 50176), lambda i: (0, 0)),
            pl.BlockSpec((1, F), lambda i: (0, 0)),
        ],
        out_specs=[
            pl.BlockSpec((R, F), lambda i: (i, 0)),
            pl.BlockSpec((8, F), lambda i: (0, 0)),
        ],
        out_shape=[
            jax.ShapeDtypeStruct((N, F), jnp.float32),
            jax.ShapeDtypeStruct((8, F), jnp.float32),
        ],
    )(agg, y, deg, b)


def _bn_relu(pre_ref, st_ref, g_ref, be_ref):
    mu = st_ref[0:1, :] / N
    var = st_ref[1:2, :] / N - mu * mu
    inv = lax.rsqrt(var + EPS)
    return jax.nn.relu((pre_ref[...] - mu) * inv * g_ref[...] + be_ref[...])


def _apply_call(pre, st, g, be, wn, deg):
    def body(pre_ref, st_ref, g_ref, be_ref, w_ref, deg_ref, y_ref):
        i = pl.program_id(0)
        h = _bn_relu(pre_ref, st_ref, g_ref, be_ref)
        dis = _dis_block(deg_ref, i)
        y_ref[...] = jnp.dot(h, w_ref[...],
                             preferred_element_type=jnp.float32) * dis

    return pl.pallas_call(
        body,
        grid=(NB,),
        in_specs=[
            pl.BlockSpec((R, F), lambda i: (i, 0)),
            pl.BlockSpec((8, F), lambda i: (0, 0)),
            pl.BlockSpec((1, F), lambda i: (0, 0)),
            pl.BlockSpec((1, F), lambda i: (0, 0)),
            pl.BlockSpec((F, F), lambda i: (0, 0)),
            pl.BlockSpec((2, 50176), lambda i: (0, 0)),
        ],
        out_specs=pl.BlockSpec((R, F), lambda i: (i, 0)),
        out_shape=jax.ShapeDtypeStruct((N, F), jnp.float32),
    )(pre, st, g, be, wn, deg)


def _pool_call(pre, st, g, be, batch2):
    def body(pre_ref, st_ref, g_ref, be_ref, b_ref, out_ref, acc, cnt):
        i = pl.program_id(0)
        h = _bn_relu(pre_ref, st_ref, g_ref, be_ref)

        @pl.when(i == 0)
        def _():
            acc[...] = jnp.zeros_like(acc)
            cnt[...] = jnp.zeros_like(cnt)

        bi = b_ref[0, 0, :]
        oh = (lax.broadcasted_iota(jnp.int32, (G, R), 0)
              == bi[None, :]).astype(jnp.float32)
        acc[...] = acc[...] + jnp.dot(oh, h,
                                      preferred_element_type=jnp.float32)
        cnt[...] = cnt[...] + jnp.sum(oh, 1, keepdims=True)
        out_ref[...] = acc[...] / jnp.maximum(cnt[...], 1.0)

    return pl.pallas_call(
        body,
        grid=(NB,),
        in_specs=[
            pl.BlockSpec((R, F), lambda i: (i, 0)),
            pl.BlockSpec((8, F), lambda i: (0, 0)),
            pl.BlockSpec((1, F), lambda i: (0, 0)),
            pl.BlockSpec((1, F), lambda i: (0, 0)),
            pl.BlockSpec((1, 1, R), lambda i: (i, 0, 0)),
        ],
        out_specs=pl.BlockSpec((G, F), lambda i: (0, 0)),
        out_shape=jax.ShapeDtypeStruct((G, F), jnp.float32),
        scratch_shapes=[
            pltpu.VMEM((G, F), jnp.float32),
            pltpu.VMEM((G, 1), jnp.float32),
        ],
    )(pre, st, g, be, batch2)


def kernel(x, edge_index, batch,
           W1, b1, g1, be1, W2, b2, g2, be2, W3, b3, g3, be3):
    f32 = jnp.float32
    src = edge_index[0]
    dst = edge_index[1]
    npad = E_PAD - E
    dst_pad = 50000 + (jnp.arange(npad, dtype=jnp.int32) % 176)
    dst2 = jnp.concatenate([dst, dst_pad]).reshape(EROWS, 128)
    src2 = jnp.concatenate(
        [src, jnp.zeros((npad,), jnp.int32)]).reshape(EROWS, 128)
    cc = jnp.full((16,), NWIN, jnp.int32)

    w1p = jnp.pad(W1, ((0, 0), (0, F - W1.shape[1]))).astype(f32)
    w2p = jnp.pad(W2, ((0, F - W2.shape[0]), (0, 0))).astype(f32)
    pad1 = lambda v: jnp.pad(v, (0, F - v.shape[0])).reshape(1, F).astype(f32)
    b1p, g1p, be1p = pad1(b1), pad1(g1), pad1(be1)
    b2p, g2p, be2p = pad1(b2), pad1(g2), pad1(be2)
    b3p, g3p, be3p = pad1(b3), pad1(g3), pad1(be3)
    batch2 = batch.reshape(NB, 1, R)

    deg = _deg_call(dst2)
    bsrc, bdst, bcnt = _bin_call(src2, dst2, cc)

    y1 = _tc1_call(x, w1p, deg)
    agg1 = _agg_call(y1, bsrc, bdst, bcnt)
    pre1, st1 = _stats_call(agg1, y1, deg, b1p)

    y2 = _apply_call(pre1, st1, g1p, be1p, w2p, deg)
    agg2 = _agg_call(y2, bsrc, bdst, bcnt)
    pre2, st2 = _stats_call(agg2, y2, deg, b2p)

    y3 = _apply_call(pre2, st2, g2p, be2p, W3.astype(f32), deg)
    agg3 = _agg_call(y3, bsrc, bdst, bcnt)
    pre3, st3 = _stats_call(agg3, y3, deg, b3p)

    return _pool_call(pre3, st3, g3p, be3p, batch2)


# speedup vs baseline: 13.1491x; 13.1491x over previous
import functools

import jax
import jax.numpy as jnp
from jax import lax
from jax.experimental import pallas as pl
from jax.experimental.pallas import tpu as pltpu
from jax.experimental.pallas import tpu_sc as plsc

N = 50000
E = 800000
G = 256
F = 128
NQ = 12512
ACC_R = 12544
DEG_R = 51200
E_PAD = 819200
EROWS = E_PAD // 128
WROWS = EROWS // 32
NWIN = WROWS // 8
REGCAP = 25600
RING = 2048
NB = 50
R = 1000
EPS = 1e-5

_SC_PARAMS = dict(
    compiler_params=pltpu.CompilerParams(needs_layout_passes=False),
)


def _sc_mesh():
    return plsc.VectorSubcoreMesh(core_axis_name="c", subcore_axis_name="s",
                                  num_cores=2, num_subcores=16)


def _pick(vmem_ref, n_entries, target):
    tgt = jnp.full((16,), target, jnp.int32)
    acc16 = jnp.zeros((16,), jnp.int32)
    for g in range(n_entries // 16):
        vg = vmem_ref[pl.ds(g * 16, 16)]
        lane = lax.iota(jnp.int32, 16) + g * 16
        acc16 = jnp.maximum(acc16, jnp.where(lane == tgt, vg, 0))
    return jnp.max(acc16, axis=0)


def _deg_call(dst2):
    @functools.partial(
        pl.kernel,
        out_type=jax.ShapeDtypeStruct((2, DEG_R), jnp.float32),
        mesh=_sc_mesh(),
        **_SC_PARAMS,
        scratch_types=[
            pltpu.VMEM((8, 128), jnp.int32),
            pltpu.VMEM((128,), jnp.float32),
            pltpu.VMEM((3200,), jnp.float32),
            pltpu.VMEM_SHARED((DEG_R,), jnp.float32),
        ],
    )
    def k(dst_ref, out, didx, ones_v, zbuf, acc):
        c = lax.axis_index("c")
        t = lax.axis_index("s")
        w = c * 16 + t
        one16 = jnp.full((16,), 1.0, jnp.float32)
        z16 = jnp.zeros((16,), jnp.float32)
        for i in range(8):
            ones_v[pl.ds(i * 16, 16)] = one16

        def zb(i, _):
            zbuf[pl.ds(i * 16, 16)] = z16
            return 0

        lax.fori_loop(0, 200, zb, 0)
        pltpu.sync_copy(zbuf, acc.at[pl.ds(t * 3200, 3200)])
        plsc.subcore_barrier()

        def win(wi, _):
            pltpu.sync_copy(dst_ref.at[pl.ds(w * WROWS + wi * 8, 8)], didx)
            for j in range(8):
                pltpu.sync_copy(ones_v, acc.at[didx.at[j]], add=True)
            return 0

        lax.fori_loop(0, NWIN, win, 0)
        plsc.subcore_barrier()
        pltpu.sync_copy(acc.at[pl.ds(t * 3200, 3200)], zbuf)
        pltpu.sync_copy(zbuf, out.at[c].at[pl.ds(t * 3200, 3200)])

    return k(dst2)


def _bin_call(src2, dst2, cc):
    @functools.partial(
        pl.kernel,
        out_type=[
            jax.ShapeDtypeStruct((4 * 32 * REGCAP,), jnp.int32),
            jax.ShapeDtypeStruct((4 * 32 * REGCAP,), jnp.int32),
            jax.ShapeDtypeStruct((32 * 128,), jnp.int32),
        ],
        mesh=_sc_mesh(),
        **_SC_PARAMS,
        scratch_types=[
            pltpu.VMEM((8, 128), jnp.int32),
            pltpu.VMEM((8, 128), jnp.int32),
            [pltpu.VMEM((RING,), jnp.int32) for _ in range(4)],
            [pltpu.VMEM((RING,), jnp.int32) for _ in range(4)],
            pltpu.VMEM((128,), jnp.int32),
            pltpu.VMEM((16,), jnp.int32),
        ],
    )
    def k(src_ref, dst_ref, cc_ref, bsrc, bdst, bcnt,
          swin, dwin, rs, rd, cbuf, ncv):
        c = lax.axis_index("c")
        t = lax.axis_index("s")
        w = c * 16 + t
        iota16 = lax.iota(jnp.int32, 16)
        pltpu.sync_copy(cc_ref, ncv)
        nwin = _pick(ncv, 16, 0)

        def win(wi, carry):
            cnts, fls = carry
            pltpu.sync_copy(src_ref.at[pl.ds(w * WROWS + wi * 8, 8)], swin)
            pltpu.sync_copy(dst_ref.at[pl.ds(w * WROWS + wi * 8, 8)], dwin)

            def grp(g, cs):
                c0, c1, c2, c3 = cs
                vd = dwin[g // 8, pl.ds((g % 8) * 16, 16)]
                vs = swin[g // 8, pl.ds((g % 8) * 16, 16)]
                b = ((vd >= NQ).astype(jnp.int32)
                     + (vd >= 2 * NQ).astype(jnp.int32)
                     + (vd >= 3 * NQ).astype(jnp.int32))
                dl = vd - b * NQ
                valid = vd < N
                outs = []
                for bi, cb in enumerate((c0, c1, c2, c3)):
                    msk = jnp.logical_and(b == bi, valid)
                    pos = plsc.cumsum(msk.astype(jnp.int32), mask=msk)
                    off = (cb + pos - 1) & (RING - 1)
                    plsc.store_scatter(rd[bi], [off], dl, mask=msk)
                    plsc.store_scatter(rs[bi], [off], vs, mask=msk)
                    npop = plsc.all_reduce_population_count(msk)
                    outs.append(cb + npop)
                return tuple(outs)

            cnts = lax.fori_loop(0, 64, grp, cnts)
            new_fls = []
            for bi in range(4):
                csc = jnp.max(cnts[bi], axis=0)
                fl = fls[bi]
                do = (csc - fl) >= 1024
                base = (bi * 32 + w) * REGCAP

                @pl.when(do)
                def _(bi=bi, fl=fl, base=base):
                    rb = pl.multiple_of(fl & (RING - 1), 1024)
                    ho = pl.multiple_of(base + fl, 512)
                    pltpu.sync_copy(rs[bi].at[pl.ds(rb, 1024)],
                                    bsrc.at[pl.ds(ho, 1024)])
                    pltpu.sync_copy(rd[bi].at[pl.ds(rb, 1024)],
                                    bdst.at[pl.ds(ho, 1024)])

                new_fls.append(jnp.where(do, fl + 1024, fl))
            return cnts, tuple(new_fls)

        z = jnp.zeros((16,), jnp.int32)
        zs = jnp.int32(0)
        (cnts, fls) = lax.fori_loop(
            0, nwin, win, ((z, z, z, z), (zs, zs, zs, zs)))

        cvec = jnp.zeros((16,), jnp.int32)
        for bi in range(4):
            csc = jnp.max(cnts[bi], axis=0)
            cpad = ((csc + 511) // 512) * 512
            cpad16 = jnp.full((16,), cpad, jnp.int32)
            for g in range(32):
                idx = csc + g * 16 + iota16
                msk = idx < cpad16
                off = idx & (RING - 1)
                plsc.store_scatter(rd[bi], [off],
                                   NQ + (idx & 31), mask=msk)
                plsc.store_scatter(rs[bi], [off],
                                   (idx * 37) & 32767, mask=msk)
            base = (bi * 32 + w) * REGCAP
            nrem = (cpad - fls[bi]) // 512

            def fin(r, _, bi=bi, base=base, fl=fls[bi]):
                off = fl + r * 512
                rb = pl.multiple_of(off & (RING - 1), 512)
                ho = pl.multiple_of(base + off, 512)
                pltpu.sync_copy(rs[bi].at[pl.ds(rb, 512)],
                                bsrc.at[pl.ds(ho, 512)])
                pltpu.sync_copy(rd[bi].at[pl.ds(rb, 512)],
                                bdst.at[pl.ds(ho, 512)])
                return 0

            lax.fori_loop(0, nrem, fin, 0)
            cvec = jnp.where(iota16 == bi, cpad, cvec)

        cbuf[pl.ds(0, 16)] = cvec
        z16i = jnp.zeros((16,), jnp.int32)
        for g in range(1, 8):
            cbuf[pl.ds(g * 16, 16)] = z16i
        pltpu.sync_copy(cbuf,
                        bcnt.at[pl.ds(pl.multiple_of(w * 128, 128), 128)])

    return k(src2, dst2, cc)


def _agg_call(y, bsrc, bdst, bcnt):
    @functools.partial(
        pl.kernel,
        out_type=jax.ShapeDtypeStruct((N, F), jnp.float32),
        mesh=_sc_mesh(),
        **_SC_PARAMS,
        scratch_types=[
            pltpu.VMEM((128,), jnp.int32),
            pltpu.VMEM((1, 128), jnp.int32),
            pltpu.VMEM((128, F), jnp.float32),
            pltpu.VMEM((28, F), jnp.float32),
            pltpu.VMEM((128,), jnp.int32),
            pltpu.VMEM_SHARED((ACC_R, F), jnp.float32),
            pltpu.SemaphoreType.DMA,
        ],
    )
    def k(y_ref, bsrc_ref, bdst_ref, bcnt_ref, out,
          sidx, didx, rows, zbuf, cslot, acc, sem):
        c = lax.axis_index("c")
        t = lax.axis_index("s")
        iota16 = lax.iota(jnp.int32, 16)
        z16 = jnp.zeros((16,), jnp.float32)

        def zb(i, _):
            zbuf[i // 8, pl.ds((i % 8) * 16, 16)] = z16
            return 0

        lax.fori_loop(0, 28 * 8, zb, 0)

        for j in range(2):
            q = 2 * c + j
            for kk in range(28):
                pltpu.sync_copy(zbuf, acc.at[pl.ds(t * 784 + kk * 28, 28)])
            plsc.subcore_barrier()

            for rj in range(2):
                w = 2 * t + rj
                pltpu.sync_copy(
                    bcnt_ref.at[pl.ds(pl.multiple_of(w * 128, 128), 128)],
                    cslot)
                cg = cslot[pl.ds(0, 16)]
                nw = jnp.max(jnp.where(iota16 == q, cg, 0), axis=0) // 128
                base = (q * 32 + w) * REGCAP

                def win(wi, _, base=base):
                    off = pl.multiple_of(base + wi * 128, 128)
                    pltpu.sync_copy(bsrc_ref.at[pl.ds(off, 128)], sidx)
                    pltpu.sync_copy(bdst_ref.at[pl.ds(off, 128)], didx.at[0])
                    pltpu.async_copy(y_ref.at[sidx], rows, sem).wait()
                    pltpu.sync_copy(rows, acc.at[didx.at[0]], add=True)
                    return 0

                lax.fori_loop(0, nw, win, 0)

            plsc.subcore_barrier()
            @pl.when(t < 15)
            def _(q=q):
                pltpu.sync_copy(acc.at[pl.ds(t * 784, 784)],
                                out.at[pl.ds(q * NQ + t * 784, 784)])

            @pl.when(jnp.logical_and(t == 15, q < 3))
            def _(q=q):
                pltpu.sync_copy(acc.at[pl.ds(15 * 784, 752)],
                                out.at[pl.ds(q * NQ + 15 * 784, 752)])

            @pl.when(jnp.logical_and(t == 15, q == 3))
            def _(q=q):
                pltpu.sync_copy(acc.at[pl.ds(15 * 784, 704)],
                                out.at[pl.ds(q * NQ + 15 * 784, 704)])

            plsc.subcore_barrier()

    return k(y, bsrc, bdst, bcnt)


def _dis_call(deg):
    def body(deg_ref, dis_ref):
        d = deg_ref[0, :] + deg_ref[1, :] + 1.0
        dis_ref[...] = lax.rsqrt(d)[None, None, :]

    return pl.pallas_call(
        body,
        grid=(DEG_R // 1024,),
        in_specs=[pl.BlockSpec((2, 1024), lambda i: (0, i))],
        out_specs=pl.BlockSpec((1, 1, 1024), lambda i: (i, 0, 0)),
        out_shape=jax.ShapeDtypeStruct((DEG_R // 1024, 1, 1024), jnp.float32),
    )(deg)


def _tc1_call(x, w1, dis3):
    def body(x_ref, w_ref, dis_ref, y_ref):
        dis = dis_ref[0, 0, :][:, None]
        xw = jnp.dot(x_ref[...], w_ref[...],
                     preferred_element_type=jnp.float32)
        y_ref[...] = xw * dis

    return pl.pallas_call(
        body,
        grid=(NB,),
        in_specs=[
            pl.BlockSpec((R, F), lambda i: (i, 0)),
            pl.BlockSpec((F, F), lambda i: (0, 0)),
            pl.BlockSpec((1, 1, R), lambda i: (i, 0, 0)),
        ],
        out_specs=pl.BlockSpec((R, F), lambda i: (i, 0)),
        out_shape=jax.ShapeDtypeStruct((N, F), jnp.float32),
    )(x, w1, dis3)


def _stats_call(agg, y, dis3, b):
    def body(agg_ref, y_ref, dis_ref, b_ref, pre_ref, st_ref):
        i = pl.program_id(0)
        dis = dis_ref[0, 0, :][:, None]
        pre = (agg_ref[...] + y_ref[...]) * dis + b_ref[...]
        pre_ref[...] = pre

        @pl.when(i == 0)
        def _():
            st_ref[...] = jnp.zeros_like(st_ref)

        st_ref[0:1, :] = st_ref[0:1, :] + jnp.sum(pre, 0, keepdims=True)
        st_ref[1:2, :] = st_ref[1:2, :] + jnp.sum(pre * pre, 0, keepdims=True)

    return pl.pallas_call(
        body,
        grid=(NB,),
        in_specs=[
            pl.BlockSpec((R, F), lambda i: (i, 0)),
            pl.BlockSpec((R, F), lambda i: (i, 0)),
            pl.BlockSpec((1, 1, R), lambda i: (i, 0, 0)),
            pl.BlockSpec((1, F), lambda i: (0, 0)),
        ],
        out_specs=[
            pl.BlockSpec((R, F), lambda i: (i, 0)),
            pl.BlockSpec((8, F), lambda i: (0, 0)),
        ],
        out_shape=[
            jax.ShapeDtypeStruct((N, F), jnp.float32),
            jax.ShapeDtypeStruct((8, F), jnp.float32),
        ],
    )(agg, y, dis3, b)


def _bn_relu(pre_ref, st_ref, g_ref, be_ref):
    mu = st_ref[0:1, :] / N
    var = st_ref[1:2, :] / N - mu * mu
    inv = lax.rsqrt(var + EPS)
    return jax.nn.relu((pre_ref[...] - mu) * inv * g_ref[...] + be_ref[...])


def _apply_call(pre, st, g, be, wn, dis3):
    def body(pre_ref, st_ref, g_ref, be_ref, w_ref, dis_ref, y_ref):
        h = _bn_relu(pre_ref, st_ref, g_ref, be_ref)
        dis = dis_ref[0, 0, :][:, None]
        y_ref[...] = jnp.dot(h, w_ref[...],
                             preferred_element_type=jnp.float32) * dis

    return pl.pallas_call(
        body,
        grid=(NB,),
        in_specs=[
            pl.BlockSpec((R, F), lambda i: (i, 0)),
            pl.BlockSpec((8, F), lambda i: (0, 0)),
            pl.BlockSpec((1, F), lambda i: (0, 0)),
            pl.BlockSpec((1, F), lambda i: (0, 0)),
            pl.BlockSpec((F, F), lambda i: (0, 0)),
            pl.BlockSpec((1, 1, R), lambda i: (i, 0, 0)),
        ],
        out_specs=pl.BlockSpec((R, F), lambda i: (i, 0)),
        out_shape=jax.ShapeDtypeStruct((N, F), jnp.float32),
    )(pre, st, g, be, wn, dis3)


def _pool_call(pre, st, g, be, batch2):
    def body(pre_ref, st_ref, g_ref, be_ref, b_ref, out_ref, acc, cnt):
        i = pl.program_id(0)
        h = _bn_relu(pre_ref, st_ref, g_ref, be_ref)

        @pl.when(i == 0)
        def _():
            acc[...] = jnp.zeros_like(acc)
            cnt[...] = jnp.zeros_like(cnt)

        bi = b_ref[0, 0, :]
        oh = (lax.broadcasted_iota(jnp.int32, (G, R), 0)
              == bi[None, :]).astype(jnp.float32)
        acc[...] = acc[...] + jnp.dot(oh, h,
                                      preferred_element_type=jnp.float32)
        cnt[...] = cnt[...] + jnp.sum(oh, 1, keepdims=True)
        out_ref[...] = acc[...] / jnp.maximum(cnt[...], 1.0)

    return pl.pallas_call(
        body,
        grid=(NB,),
        in_specs=[
            pl.BlockSpec((R, F), lambda i: (i, 0)),
            pl.BlockSpec((8, F), lambda i: (0, 0)),
            pl.BlockSpec((1, F), lambda i: (0, 0)),
            pl.BlockSpec((1, F), lambda i: (0, 0)),
            pl.BlockSpec((1, 1, R), lambda i: (i, 0, 0)),
        ],
        out_specs=pl.BlockSpec((G, F), lambda i: (0, 0)),
        out_shape=jax.ShapeDtypeStruct((G, F), jnp.float32),
        scratch_shapes=[
            pltpu.VMEM((G, F), jnp.float32),
            pltpu.VMEM((G, 1), jnp.float32),
        ],
    )(pre, st, g, be, batch2)


def kernel(x, edge_index, batch,
           W1, b1, g1, be1, W2, b2, g2, be2, W3, b3, g3, be3):
    f32 = jnp.float32
    src = edge_index[0]
    dst = edge_index[1]
    npad = E_PAD - E
    dst_pad = 50000 + (jnp.arange(npad, dtype=jnp.int32) % 1176)
    dst2 = jnp.concatenate([dst, dst_pad]).reshape(EROWS, 128)
    src2 = jnp.concatenate(
        [src, jnp.zeros((npad,), jnp.int32)]).reshape(EROWS, 128)
    cc = jnp.full((16,), NWIN, jnp.int32)

    w1p = jnp.pad(W1, ((0, 0), (0, F - W1.shape[1]))).astype(f32)
    w2p = jnp.pad(W2, ((0, F - W2.shape[0]), (0, 0))).astype(f32)
    pad1 = lambda v: jnp.pad(v, (0, F - v.shape[0])).reshape(1, F).astype(f32)
    b1p, g1p, be1p = pad1(b1), pad1(g1), pad1(be1)
    b2p, g2p, be2p = pad1(b2), pad1(g2), pad1(be2)
    b3p, g3p, be3p = pad1(b3), pad1(g3), pad1(be3)
    batch2 = batch.reshape(NB, 1, R)

    deg = _deg_call(dst2)
    bsrc, bdst, bcnt = _bin_call(src2, dst2, cc)
    dis3 = _dis_call(deg).reshape(DEG_R)[:N].reshape(NB, 1, R)

    y1 = _tc1_call(x, w1p, dis3)
    agg1 = _agg_call(y1, bsrc, bdst, bcnt)
    pre1, st1 = _stats_call(agg1, y1, dis3, b1p)

    y2 = _apply_call(pre1, st1, g1p, be1p, w2p, dis3)
    agg2 = _agg_call(y2, bsrc, bdst, bcnt)
    pre2, st2 = _stats_call(agg2, y2, dis3, b2p)

    y3 = _apply_call(pre2, st2, g2p, be2p, W3.astype(f32), dis3)
    agg3 = _agg_call(y3, bsrc, bdst, bcnt)
    pre3, st3 = _stats_call(agg3, y3, dis3, b3p)

    return _pool_call(pre3, st3, g3p, be3p, batch2)


# speedup vs baseline: 17.4674x; 1.3284x over previous
import functools

import jax
import jax.numpy as jnp
from jax import lax
from jax.experimental import pallas as pl
from jax.experimental.pallas import tpu as pltpu
from jax.experimental.pallas import tpu_sc as plsc

N = 50000
E = 800000
G = 256
F = 128
NQ = 6272
NBKT = 8
ACC_R = 6400
DEG_R = 51200
E_PAD = 819200
EROWS = E_PAD // 128
WROWS = EROWS // 32
NWIN = WROWS // 8
REGCAP = 25600
RING = 2048
NB = 50
R = 1000
EPS = 1e-5

_SC_PARAMS = dict(
    compiler_params=pltpu.CompilerParams(needs_layout_passes=False),
)


def _sc_mesh():
    return plsc.VectorSubcoreMesh(core_axis_name="c", subcore_axis_name="s",
                                  num_cores=2, num_subcores=16)


def _pick(vmem_ref, n_entries, target):
    tgt = jnp.full((16,), target, jnp.int32)
    acc16 = jnp.zeros((16,), jnp.int32)
    for g in range(n_entries // 16):
        vg = vmem_ref[pl.ds(g * 16, 16)]
        lane = lax.iota(jnp.int32, 16) + g * 16
        acc16 = jnp.maximum(acc16, jnp.where(lane == tgt, vg, 0))
    return jnp.max(acc16, axis=0)


def _deg_call(dst2):
    @functools.partial(
        pl.kernel,
        out_type=jax.ShapeDtypeStruct((2, DEG_R), jnp.float32),
        mesh=_sc_mesh(),
        **_SC_PARAMS,
        scratch_types=[
            pltpu.VMEM((8, 128), jnp.int32),
            pltpu.VMEM((128,), jnp.float32),
            pltpu.VMEM((3200,), jnp.float32),
            pltpu.VMEM_SHARED((DEG_R,), jnp.float32),
        ],
    )
    def k(dst_ref, out, didx, ones_v, zbuf, acc):
        c = lax.axis_index("c")
        t = lax.axis_index("s")
        w = c * 16 + t
        one16 = jnp.full((16,), 1.0, jnp.float32)
        z16 = jnp.zeros((16,), jnp.float32)
        for i in range(8):
            ones_v[pl.ds(i * 16, 16)] = one16

        def zb(i, _):
            zbuf[pl.ds(i * 16, 16)] = z16
            return 0

        lax.fori_loop(0, 200, zb, 0)
        pltpu.sync_copy(zbuf, acc.at[pl.ds(t * 3200, 3200)])
        plsc.subcore_barrier()

        def win(wi, _):
            pltpu.sync_copy(dst_ref.at[pl.ds(w * WROWS + wi * 8, 8)], didx)
            for j in range(8):
                pltpu.sync_copy(ones_v, acc.at[didx.at[j]], add=True)
            return 0

        lax.fori_loop(0, NWIN, win, 0)
        plsc.subcore_barrier()
        pltpu.sync_copy(acc.at[pl.ds(t * 3200, 3200)], zbuf)
        pltpu.sync_copy(zbuf, out.at[c].at[pl.ds(t * 3200, 3200)])

    return k(dst2)


def _bin_call(src2, dst2, cc):
    @functools.partial(
        pl.kernel,
        out_type=[
            jax.ShapeDtypeStruct((NBKT * 32 * REGCAP,), jnp.int32),
            jax.ShapeDtypeStruct((NBKT * 32 * REGCAP,), jnp.int32),
            jax.ShapeDtypeStruct((32 * 128,), jnp.int32),
        ],
        mesh=_sc_mesh(),
        **_SC_PARAMS,
        scratch_types=[
            pltpu.VMEM((8, 128), jnp.int32),
            pltpu.VMEM((8, 128), jnp.int32),
            [pltpu.VMEM((RING,), jnp.int32) for _ in range(NBKT)],
            [pltpu.VMEM((RING,), jnp.int32) for _ in range(NBKT)],
            pltpu.VMEM((128,), jnp.int32),
            pltpu.VMEM((16,), jnp.int32),
        ],
    )
    def k(src_ref, dst_ref, cc_ref, bsrc, bdst, bcnt,
          swin, dwin, rs, rd, cbuf, ncv):
        c = lax.axis_index("c")
        t = lax.axis_index("s")
        w = c * 16 + t
        iota16 = lax.iota(jnp.int32, 16)
        pltpu.sync_copy(cc_ref, ncv)
        nwin = _pick(ncv, 16, 0)

        def win(wi, carry):
            cnts, fls = carry
            pltpu.sync_copy(src_ref.at[pl.ds(w * WROWS + wi * 8, 8)], swin)
            pltpu.sync_copy(dst_ref.at[pl.ds(w * WROWS + wi * 8, 8)], dwin)

            def grp(g, cs):
                vd = dwin[g // 8, pl.ds((g % 8) * 16, 16)]
                vs = swin[g // 8, pl.ds((g % 8) * 16, 16)]
                b = (vd >= NQ).astype(jnp.int32)
                for kq in range(2, NBKT):
                    b = b + (vd >= kq * NQ).astype(jnp.int32)
                dl = vd - b * NQ
                valid = vd < N
                outs = []
                for bi, cb in enumerate(cs):
                    msk = jnp.logical_and(b == bi, valid)
                    pos = plsc.cumsum(msk.astype(jnp.int32), mask=msk)
                    off = (cb + pos - 1) & (RING - 1)
                    plsc.store_scatter(rd[bi], [off], dl, mask=msk)
                    plsc.store_scatter(rs[bi], [off], vs, mask=msk)
                    npop = plsc.all_reduce_population_count(msk)
                    outs.append(cb + npop)
                return tuple(outs)

            cnts = lax.fori_loop(0, 64, grp, cnts)
            new_fls = []
            for bi in range(NBKT):
                csc = jnp.max(cnts[bi], axis=0)
                fl = fls[bi]
                do = (csc - fl) >= 1024
                base = (bi * 32 + w) * REGCAP

                @pl.when(do)
                def _(bi=bi, fl=fl, base=base):
                    rb = pl.multiple_of(fl & (RING - 1), 1024)
                    ho = pl.multiple_of(base + fl, 512)
                    pltpu.sync_copy(rs[bi].at[pl.ds(rb, 1024)],
                                    bsrc.at[pl.ds(ho, 1024)])
                    pltpu.sync_copy(rd[bi].at[pl.ds(rb, 1024)],
                                    bdst.at[pl.ds(ho, 1024)])

                new_fls.append(jnp.where(do, fl + 1024, fl))
            return cnts, tuple(new_fls)

        z = jnp.zeros((16,), jnp.int32)
        zs = jnp.int32(0)
        (cnts, fls) = lax.fori_loop(
            0, nwin, win, (tuple(z for _ in range(NBKT)),
                           tuple(zs for _ in range(NBKT))))

        cvec = jnp.zeros((16,), jnp.int32)
        for bi in range(NBKT):
            csc = jnp.max(cnts[bi], axis=0)
            cpad = ((csc + 511) // 512) * 512
            cpad16 = jnp.full((16,), cpad, jnp.int32)
            for g in range(32):
                idx = csc + g * 16 + iota16
                msk = idx < cpad16
                off = idx & (RING - 1)
                plsc.store_scatter(rd[bi], [off],
                                   NQ + (idx & 31), mask=msk)
                plsc.store_scatter(rs[bi], [off],
                                   (idx * 37) & 32767, mask=msk)
            base = (bi * 32 + w) * REGCAP
            nrem = (cpad - fls[bi]) // 512

            def fin(r, _, bi=bi, base=base, fl=fls[bi]):
                off = fl + r * 512
                rb = pl.multiple_of(off & (RING - 1), 512)
                ho = pl.multiple_of(base + off, 512)
                pltpu.sync_copy(rs[bi].at[pl.ds(rb, 512)],
                                bsrc.at[pl.ds(ho, 512)])
                pltpu.sync_copy(rd[bi].at[pl.ds(rb, 512)],
                                bdst.at[pl.ds(ho, 512)])
                return 0

            lax.fori_loop(0, nrem, fin, 0)
            cvec = jnp.where(iota16 == bi, cpad, cvec)

        cbuf[pl.ds(0, 16)] = cvec
        z16i = jnp.zeros((16,), jnp.int32)
        for g in range(1, 8):
            cbuf[pl.ds(g * 16, 16)] = z16i
        pltpu.sync_copy(cbuf,
                        bcnt.at[pl.ds(pl.multiple_of(w * 128, 128), 128)])

    return k(src2, dst2, cc)


def _agg_call(y, bsrc, bdst, bcnt):
    @functools.partial(
        pl.kernel,
        out_type=jax.ShapeDtypeStruct((N, F), jnp.float32),
        mesh=_sc_mesh(),
        **_SC_PARAMS,
        scratch_types=[
            pltpu.VMEM((4, 128), jnp.int32),
            pltpu.VMEM((4, 128), jnp.int32),
            pltpu.VMEM((2, 256, F), jnp.float32),
            pltpu.VMEM((80, F), jnp.float32),
            pltpu.VMEM((128,), jnp.int32),
            pltpu.VMEM_SHARED((ACC_R, F), jnp.float32),
            pltpu.SemaphoreType.DMA,
            pltpu.SemaphoreType.DMA,
        ],
    )
    def k(y_ref, bsrc_ref, bdst_ref, bcnt_ref, out,
          sidx, didx, rows, zbuf, cslot, acc, semA, semB):
        c = lax.axis_index("c")
        t = lax.axis_index("s")
        iota16 = lax.iota(jnp.int32, 16)
        z16 = jnp.zeros((16,), jnp.float32)
        sems = (semA, semB)

        def zb(i, _):
            zbuf[i // 8, pl.ds((i % 8) * 16, 16)] = z16
            return 0

        lax.fori_loop(0, 80 * 8, zb, 0)

        def stage(p, off):
            off = pl.multiple_of(off, 128)
            pltpu.sync_copy(bsrc_ref.at[pl.ds(off, 128)], sidx.at[2 * p])
            pltpu.sync_copy(bsrc_ref.at[pl.ds(off + 128, 128)],
                            sidx.at[2 * p + 1])
            pltpu.sync_copy(bdst_ref.at[pl.ds(off, 128)], didx.at[2 * p])
            pltpu.sync_copy(bdst_ref.at[pl.ds(off + 128, 128)],
                            didx.at[2 * p + 1])

        def fire(p):
            for jj in range(2):
                pltpu.async_copy(y_ref.at[sidx.at[2 * p + jj]],
                                 rows.at[p].at[pl.ds(jj * 128, 128)],
                                 sems[p])

        def wait_gather(p):
            for jj in range(2):
                pltpu.make_async_copy(
                    y_ref.at[sidx.at[2 * p + jj]],
                    rows.at[p].at[pl.ds(jj * 128, 128)],
                    sems[p]).wait()

        def scat(p):
            for jj in range(2):
                pltpu.sync_copy(rows.at[p].at[pl.ds(jj * 128, 128)],
                                acc.at[didx.at[2 * p + jj]], add=True)

        for j in range(4):
            q = 4 * c + j
            for kk in range(5):
                pltpu.sync_copy(zbuf, acc.at[pl.ds(t * 400 + kk * 80, 80)])
            plsc.subcore_barrier()

            for rj in range(2):
                w = 2 * t + rj
                pltpu.sync_copy(
                    bcnt_ref.at[pl.ds(pl.multiple_of(w * 128, 128), 128)],
                    cslot)
                cg = cslot[pl.ds(0, 16)]
                cnt = jnp.max(jnp.where(iota16 == q, cg, 0), axis=0)
                npair = cnt // 512
                base = (q * 32 + w) * REGCAP

                @pl.when(npair > 0)
                def _(base=base):
                    stage(0, base)
                    fire(0)

                def pair(k2, _, base=base, npair=npair):
                    stage(1, base + (2 * k2 + 1) * 256)
                    fire(1)
                    wait_gather(0)
                    scat(0)

                    @pl.when(k2 + 1 < npair)
                    def _():
                        stage(0, base + (2 * k2 + 2) * 256)
                        fire(0)

                    wait_gather(1)
                    scat(1)
                    return 0

                lax.fori_loop(0, npair, pair, 0)

            plsc.subcore_barrier()
            @pl.when(jnp.logical_or(t < 15, q < 7))
            def _(q=q):
                pltpu.sync_copy(acc.at[pl.ds(t * 392, 392)],
                                out.at[pl.ds(q * NQ + t * 392, 392)])

            @pl.when(jnp.logical_and(t == 15, q == 7))
            def _(q=q):
                pltpu.sync_copy(acc.at[pl.ds(15 * 392, 216)],
                                out.at[pl.ds(q * NQ + 15 * 392, 216)])

            plsc.subcore_barrier()

    return k(y, bsrc, bdst, bcnt)


def _dis_call(deg):
    def body(deg_ref, dis_ref):
        d = deg_ref[0, :] + deg_ref[1, :] + 1.0
        dis_ref[...] = lax.rsqrt(d)[None, None, :]

    return pl.pallas_call(
        body,
        grid=(DEG_R // 1024,),
        in_specs=[pl.BlockSpec((2, 1024), lambda i: (0, i))],
        out_specs=pl.BlockSpec((1, 1, 1024), lambda i: (i, 0, 0)),
        out_shape=jax.ShapeDtypeStruct((DEG_R // 1024, 1, 1024), jnp.float32),
    )(deg)


def _tc1_call(x, w1, dis3):
    def body(x_ref, w_ref, dis_ref, y_ref):
        dis = dis_ref[0, 0, :][:, None]
        xw = jnp.dot(x_ref[...], w_ref[...],
                     preferred_element_type=jnp.float32)
        y_ref[...] = xw * dis

    return pl.pallas_call(
        body,
        grid=(NB,),
        in_specs=[
            pl.BlockSpec((R, F), lambda i: (i, 0)),
            pl.BlockSpec((F, F), lambda i: (0, 0)),
            pl.BlockSpec((1, 1, R), lambda i: (i, 0, 0)),
        ],
        out_specs=pl.BlockSpec((R, F), lambda i: (i, 0)),
        out_shape=jax.ShapeDtypeStruct((N, F), jnp.float32),
    )(x, w1, dis3)


def _stats_call(agg, y, dis3, b):
    def body(agg_ref, y_ref, dis_ref, b_ref, pre_ref, st_ref):
        i = pl.program_id(0)
        dis = dis_ref[0, 0, :][:, None]
        pre = (agg_ref[...] + y_ref[...]) * dis + b_ref[...]
        pre_ref[...] = pre

        @pl.when(i == 0)
        def _():
            st_ref[...] = jnp.zeros_like(st_ref)

        st_ref[0:1, :] = st_ref[0:1, :] + jnp.sum(pre, 0, keepdims=True)
        st_ref[1:2, :] = st_ref[1:2, :] + jnp.sum(pre * pre, 0, keepdims=True)

    return pl.pallas_call(
        body,
        grid=(NB,),
        in_specs=[
            pl.BlockSpec((R, F), lambda i: (i, 0)),
            pl.BlockSpec((R, F), lambda i: (i, 0)),
            pl.BlockSpec((1, 1, R), lambda i: (i, 0, 0)),
            pl.BlockSpec((1, F), lambda i: (0, 0)),
        ],
        out_specs=[
            pl.BlockSpec((R, F), lambda i: (i, 0)),
            pl.BlockSpec((8, F), lambda i: (0, 0)),
        ],
        out_shape=[
            jax.ShapeDtypeStruct((N, F), jnp.float32),
            jax.ShapeDtypeStruct((8, F), jnp.float32),
        ],
    )(agg, y, dis3, b)


def _bn_relu(pre_ref, st_ref, g_ref, be_ref):
    mu = st_ref[0:1, :] / N
    var = st_ref[1:2, :] / N - mu * mu
    inv = lax.rsqrt(var + EPS)
    return jax.nn.relu((pre_ref[...] - mu) * inv * g_ref[...] + be_ref[...])


def _apply_call(pre, st, g, be, wn, dis3):
    def body(pre_ref, st_ref, g_ref, be_ref, w_ref, dis_ref, y_ref):
        h = _bn_relu(pre_ref, st_ref, g_ref, be_ref)
        dis = dis_ref[0, 0, :][:, None]
        y_ref[...] = jnp.dot(h, w_ref[...],
                             preferred_element_type=jnp.float32) * dis

    return pl.pallas_call(
        body,
        grid=(NB,),
        in_specs=[
            pl.BlockSpec((R, F), lambda i: (i, 0)),
            pl.BlockSpec((8, F), lambda i: (0, 0)),
            pl.BlockSpec((1, F), lambda i: (0, 0)),
            pl.BlockSpec((1, F), lambda i: (0, 0)),
            pl.BlockSpec((F, F), lambda i: (0, 0)),
            pl.BlockSpec((1, 1, R), lambda i: (i, 0, 0)),
        ],
        out_specs=pl.BlockSpec((R, F), lambda i: (i, 0)),
        out_shape=jax.ShapeDtypeStruct((N, F), jnp.float32),
    )(pre, st, g, be, wn, dis3)


def _pool_call(pre, st, g, be, batch2):
    def body(pre_ref, st_ref, g_ref, be_ref, b_ref, out_ref, acc, cnt):
        i = pl.program_id(0)
        h = _bn_relu(pre_ref, st_ref, g_ref, be_ref)

        @pl.when(i == 0)
        def _():
            acc[...] = jnp.zeros_like(acc)
            cnt[...] = jnp.zeros_like(cnt)

        bi = b_ref[0, 0, :]
        oh = (lax.broadcasted_iota(jnp.int32, (G, R), 0)
              == bi[None, :]).astype(jnp.float32)
        acc[...] = acc[...] + jnp.dot(oh, h,
                                      preferred_element_type=jnp.float32)
        cnt[...] = cnt[...] + jnp.sum(oh, 1, keepdims=True)
        out_ref[...] = acc[...] / jnp.maximum(cnt[...], 1.0)

    return pl.pallas_call(
        body,
        grid=(NB,),
        in_specs=[
            pl.BlockSpec((R, F), lambda i: (i, 0)),
            pl.BlockSpec((8, F), lambda i: (0, 0)),
            pl.BlockSpec((1, F), lambda i: (0, 0)),
            pl.BlockSpec((1, F), lambda i: (0, 0)),
            pl.BlockSpec((1, 1, R), lambda i: (i, 0, 0)),
        ],
        out_specs=pl.BlockSpec((G, F), lambda i: (0, 0)),
        out_shape=jax.ShapeDtypeStruct((G, F), jnp.float32),
        scratch_shapes=[
            pltpu.VMEM((G, F), jnp.float32),
            pltpu.VMEM((G, 1), jnp.float32),
        ],
    )(pre, st, g, be, batch2)


def kernel(x, edge_index, batch,
           W1, b1, g1, be1, W2, b2, g2, be2, W3, b3, g3, be3):
    f32 = jnp.float32
    src = edge_index[0]
    dst = edge_index[1]
    npad = E_PAD - E
    dst_pad = 50000 + (jnp.arange(npad, dtype=jnp.int32) % 1176)
    dst2 = jnp.concatenate([dst, dst_pad]).reshape(EROWS, 128)
    src2 = jnp.concatenate(
        [src, jnp.zeros((npad,), jnp.int32)]).reshape(EROWS, 128)
    cc = jnp.full((16,), NWIN, jnp.int32)

    w1p = jnp.pad(W1, ((0, 0), (0, F - W1.shape[1]))).astype(f32)
    w2p = jnp.pad(W2, ((0, F - W2.shape[0]), (0, 0))).astype(f32)
    pad1 = lambda v: jnp.pad(v, (0, F - v.shape[0])).reshape(1, F).astype(f32)
    b1p, g1p, be1p = pad1(b1), pad1(g1), pad1(be1)
    b2p, g2p, be2p = pad1(b2), pad1(g2), pad1(be2)
    b3p, g3p, be3p = pad1(b3), pad1(g3), pad1(be3)
    batch2 = batch.reshape(NB, 1, R)

    deg = _deg_call(dst2)
    bsrc, bdst, bcnt = _bin_call(src2, dst2, cc)
    dis3 = _dis_call(deg).reshape(DEG_R)[:N].reshape(NB, 1, R)

    y1 = _tc1_call(x, w1p, dis3)
    agg1 = _agg_call(y1, bsrc, bdst, bcnt)
    pre1, st1 = _stats_call(agg1, y1, dis3, b1p)

    y2 = _apply_call(pre1, st1, g1p, be1p, w2p, dis3)
    agg2 = _agg_call(y2, bsrc, bdst, bcnt)
    pre2, st2 = _stats_call(agg2, y2, dis3, b2p)

    y3 = _apply_call(pre2, st2, g2p, be2p, W3.astype(f32), dis3)
    agg3 = _agg_call(y3, bsrc, bdst, bcnt)
    pre3, st3 = _stats_call(agg3, y3, dis3, b3p)

    return _pool_call(pre3, st3, g3p, be3p, batch2)


# speedup vs baseline: 21.5248x; 1.2323x over previous
import functools

import jax
import jax.numpy as jnp
from jax import lax
from jax.experimental import pallas as pl
from jax.experimental.pallas import tpu as pltpu
from jax.experimental.pallas import tpu_sc as plsc

N = 50000
E = 800000
G = 256
F = 128
NQ = 6272
NBKT = 8
ACC_R = 6400
DEG_R = 51200
E_PAD = 819200
EROWS = E_PAD // 128
WROWS = EROWS // 32
NWIN = WROWS // 8
REGCAP = 25600
RING = 2048
NB = 50
R = 1000
EPS = 1e-5

_SC_PARAMS = dict(
    compiler_params=pltpu.CompilerParams(needs_layout_passes=False),
)


def _sc_mesh():
    return plsc.VectorSubcoreMesh(core_axis_name="c", subcore_axis_name="s",
                                  num_cores=2, num_subcores=16)


def _pick(vmem_ref, n_entries, target):
    tgt = jnp.full((16,), target, jnp.int32)
    acc16 = jnp.zeros((16,), jnp.int32)
    for g in range(n_entries // 16):
        vg = vmem_ref[pl.ds(g * 16, 16)]
        lane = lax.iota(jnp.int32, 16) + g * 16
        acc16 = jnp.maximum(acc16, jnp.where(lane == tgt, vg, 0))
    return jnp.max(acc16, axis=0)


def _deg_call(dst2):
    @functools.partial(
        pl.kernel,
        out_type=jax.ShapeDtypeStruct((2, DEG_R), jnp.float32),
        mesh=_sc_mesh(),
        **_SC_PARAMS,
        scratch_types=[
            pltpu.VMEM((8, 128), jnp.int32),
            pltpu.VMEM((128,), jnp.float32),
            pltpu.VMEM((3200,), jnp.float32),
            pltpu.VMEM_SHARED((DEG_R,), jnp.float32),
        ],
    )
    def k(dst_ref, out, didx, ones_v, zbuf, acc):
        c = lax.axis_index("c")
        t = lax.axis_index("s")
        w = c * 16 + t
        one16 = jnp.full((16,), 1.0, jnp.float32)
        z16 = jnp.zeros((16,), jnp.float32)
        for i in range(8):
            ones_v[pl.ds(i * 16, 16)] = one16

        def zb(i, _):
            zbuf[pl.ds(i * 16, 16)] = z16
            return 0

        lax.fori_loop(0, 200, zb, 0)
        pltpu.sync_copy(zbuf, acc.at[pl.ds(t * 3200, 3200)])
        plsc.subcore_barrier()

        def win(wi, _):
            pltpu.sync_copy(dst_ref.at[pl.ds(w * WROWS + wi * 8, 8)], didx)
            for j in range(8):
                pltpu.sync_copy(ones_v, acc.at[didx.at[j]], add=True)
            return 0

        lax.fori_loop(0, NWIN, win, 0)
        plsc.subcore_barrier()
        pltpu.sync_copy(acc.at[pl.ds(t * 3200, 3200)], zbuf)
        pltpu.sync_copy(zbuf, out.at[c].at[pl.ds(t * 3200, 3200)])

    return k(dst2)


def _bin_call(src2, dst2, cc):
    @functools.partial(
        pl.kernel,
        out_type=[
            jax.ShapeDtypeStruct((NBKT * 32 * REGCAP,), jnp.int32),
            jax.ShapeDtypeStruct((NBKT * 32 * REGCAP,), jnp.int32),
            jax.ShapeDtypeStruct((32 * 128,), jnp.int32),
        ],
        mesh=_sc_mesh(),
        **_SC_PARAMS,
        scratch_types=[
            pltpu.VMEM((8, 128), jnp.int32),
            pltpu.VMEM((8, 128), jnp.int32),
            [pltpu.VMEM((RING,), jnp.int32) for _ in range(NBKT)],
            [pltpu.VMEM((RING,), jnp.int32) for _ in range(NBKT)],
            pltpu.VMEM((128,), jnp.int32),
            pltpu.VMEM((16,), jnp.int32),
        ],
    )
    def k(src_ref, dst_ref, cc_ref, bsrc, bdst, bcnt,
          swin, dwin, rs, rd, cbuf, ncv):
        c = lax.axis_index("c")
        t = lax.axis_index("s")
        w = c * 16 + t
        iota16 = lax.iota(jnp.int32, 16)
        pltpu.sync_copy(cc_ref, ncv)
        nwin = _pick(ncv, 16, 0)

        def win(wi, carry):
            cnts, fls = carry
            pltpu.sync_copy(src_ref.at[pl.ds(w * WROWS + wi * 8, 8)], swin)
            pltpu.sync_copy(dst_ref.at[pl.ds(w * WROWS + wi * 8, 8)], dwin)

            def grp(g, cs):
                vd = dwin[g // 8, pl.ds((g % 8) * 16, 16)]
                vs = swin[g // 8, pl.ds((g % 8) * 16, 16)]
                b = (vd >= NQ).astype(jnp.int32)
                for kq in range(2, NBKT):
                    b = b + (vd >= kq * NQ).astype(jnp.int32)
                dl = vd - b * NQ
                valid = vd < N
                outs = []
                for bi, cb in enumerate(cs):
                    msk = jnp.logical_and(b == bi, valid)
                    pos = plsc.cumsum(msk.astype(jnp.int32), mask=msk)
                    off = (cb + pos - 1) & (RING - 1)
                    plsc.store_scatter(rd[bi], [off], dl, mask=msk)
                    plsc.store_scatter(rs[bi], [off], vs, mask=msk)
                    npop = plsc.all_reduce_population_count(msk)
                    outs.append(cb + npop)
                return tuple(outs)

            cnts = lax.fori_loop(0, 64, grp, cnts)
            new_fls = []
            for bi in range(NBKT):
                csc = jnp.max(cnts[bi], axis=0)
                fl = fls[bi]
                do = (csc - fl) >= 1024
                base = (bi * 32 + w) * REGCAP

                @pl.when(do)
                def _(bi=bi, fl=fl, base=base):
                    rb = pl.multiple_of(fl & (RING - 1), 1024)
                    ho = pl.multiple_of(base + fl, 512)
                    pltpu.sync_copy(rs[bi].at[pl.ds(rb, 1024)],
                                    bsrc.at[pl.ds(ho, 1024)])
                    pltpu.sync_copy(rd[bi].at[pl.ds(rb, 1024)],
                                    bdst.at[pl.ds(ho, 1024)])

                new_fls.append(jnp.where(do, fl + 1024, fl))
            return cnts, tuple(new_fls)

        z = jnp.zeros((16,), jnp.int32)
        zs = jnp.int32(0)
        (cnts, fls) = lax.fori_loop(
            0, nwin, win, (tuple(z for _ in range(NBKT)),
                           tuple(zs for _ in range(NBKT))))

        cvec = jnp.zeros((16,), jnp.int32)
        for bi in range(NBKT):
            csc = jnp.max(cnts[bi], axis=0)
            cpad = ((csc + 511) // 512) * 512
            cpad16 = jnp.full((16,), cpad, jnp.int32)
            for g in range(32):
                idx = csc + g * 16 + iota16
                msk = idx < cpad16
                off = idx & (RING - 1)
                plsc.store_scatter(rd[bi], [off],
                                   NQ + (idx & 31), mask=msk)
                plsc.store_scatter(rs[bi], [off],
                                   (idx * 37) & 32767, mask=msk)
            base = (bi * 32 + w) * REGCAP
            nrem = (cpad - fls[bi]) // 512

            def fin(r, _, bi=bi, base=base, fl=fls[bi]):
                off = fl + r * 512
                rb = pl.multiple_of(off & (RING - 1), 512)
                ho = pl.multiple_of(base + off, 512)
                pltpu.sync_copy(rs[bi].at[pl.ds(rb, 512)],
                                bsrc.at[pl.ds(ho, 512)])
                pltpu.sync_copy(rd[bi].at[pl.ds(rb, 512)],
                                bdst.at[pl.ds(ho, 512)])
                return 0

            lax.fori_loop(0, nrem, fin, 0)
            cvec = jnp.where(iota16 == bi, cpad, cvec)

        cbuf[pl.ds(0, 16)] = cvec
        z16i = jnp.zeros((16,), jnp.int32)
        for g in range(1, 8):
            cbuf[pl.ds(g * 16, 16)] = z16i
        pltpu.sync_copy(cbuf,
                        bcnt.at[pl.ds(pl.multiple_of(w * 128, 128), 128)])

    return k(src2, dst2, cc)


def _agg_call(y, bsrc, bdst, bcnt):
    @functools.partial(
        pl.kernel,
        out_type=jax.ShapeDtypeStruct((N, F), jnp.float32),
        mesh=_sc_mesh(),
        **_SC_PARAMS,
        scratch_types=[
            pltpu.VMEM((4, 128), jnp.int32),
            pltpu.VMEM((4, 128), jnp.int32),
            pltpu.VMEM((2, 256, F), jnp.float32),
            pltpu.VMEM((80, F), jnp.float32),
            pltpu.VMEM((128,), jnp.int32),
            pltpu.VMEM_SHARED((ACC_R, F), jnp.float32),
            pltpu.SemaphoreType.DMA,
            pltpu.SemaphoreType.DMA,
            pltpu.SemaphoreType.DMA,
            pltpu.SemaphoreType.DMA,
        ],
    )
    def k(y_ref, bsrc_ref, bdst_ref, bcnt_ref, out,
          sidx, didx, rows, zbuf, cslot, acc, semA, semB, semI, semS):
        c = lax.axis_index("c")
        t = lax.axis_index("s")
        iota16 = lax.iota(jnp.int32, 16)
        z16 = jnp.zeros((16,), jnp.float32)
        sems = (semA, semB)

        def zb(i, _):
            zbuf[i // 8, pl.ds((i % 8) * 16, 16)] = z16
            return 0

        lax.fori_loop(0, 80 * 8, zb, 0)

        def stage(p, off):
            off = pl.multiple_of(off, 128)
            cps = []
            for jj in range(2):
                cps.append(pltpu.async_copy(
                    bsrc_ref.at[pl.ds(off + jj * 128, 128)],
                    sidx.at[2 * p + jj], semI))
                cps.append(pltpu.async_copy(
                    bdst_ref.at[pl.ds(off + jj * 128, 128)],
                    didx.at[2 * p + jj], semI))
            for cp in cps:
                cp.wait()

        def fire(p):
            for jj in range(2):
                pltpu.async_copy(y_ref.at[sidx.at[2 * p + jj]],
                                 rows.at[p].at[pl.ds(jj * 128, 128)],
                                 sems[p])

        def wait_gather(p):
            for jj in range(2):
                pltpu.make_async_copy(
                    y_ref.at[sidx.at[2 * p + jj]],
                    rows.at[p].at[pl.ds(jj * 128, 128)],
                    sems[p]).wait()

        def scat(p):
            cps = [
                pltpu.async_copy(rows.at[p].at[pl.ds(jj * 128, 128)],
                                 acc.at[didx.at[2 * p + jj]], semS, add=True)
                for jj in range(2)
            ]
            for cp in cps:
                cp.wait()

        for j in range(4):
            q = 4 * c + j
            for kk in range(5):
                pltpu.sync_copy(zbuf, acc.at[pl.ds(t * 400 + kk * 80, 80)])
            plsc.subcore_barrier()

            for rj in range(2):
                w = 2 * t + rj
                pltpu.sync_copy(
                    bcnt_ref.at[pl.ds(pl.multiple_of(w * 128, 128), 128)],
                    cslot)
                cg = cslot[pl.ds(0, 16)]
                cnt = jnp.max(jnp.where(iota16 == q, cg, 0), axis=0)
                npair = cnt // 512
                base = (q * 32 + w) * REGCAP

                @pl.when(npair > 0)
                def _(base=base):
                    stage(0, base)
                    fire(0)

                def pair(k2, _, base=base, npair=npair):
                    stage(1, base + (2 * k2 + 1) * 256)
                    fire(1)
                    wait_gather(0)
                    scat(0)

                    @pl.when(k2 + 1 < npair)
                    def _():
                        stage(0, base + (2 * k2 + 2) * 256)
                        fire(0)

                    wait_gather(1)
                    scat(1)
                    return 0

                lax.fori_loop(0, npair, pair, 0)

            plsc.subcore_barrier()
            @pl.when(jnp.logical_or(t < 15, q < 7))
            def _(q=q):
                pltpu.sync_copy(acc.at[pl.ds(t * 392, 392)],
                                out.at[pl.ds(q * NQ + t * 392, 392)])

            @pl.when(jnp.logical_and(t == 15, q == 7))
            def _(q=q):
                pltpu.sync_copy(acc.at[pl.ds(15 * 392, 216)],
                                out.at[pl.ds(q * NQ + 15 * 392, 216)])

            plsc.subcore_barrier()

    return k(y, bsrc, bdst, bcnt)


def _dis_call(deg):
    def body(deg_ref, dis_ref):
        d = deg_ref[0, :] + deg_ref[1, :] + 1.0
        dis_ref[...] = lax.rsqrt(d)[None, None, :]

    return pl.pallas_call(
        body,
        grid=(DEG_R // 1024,),
        in_specs=[pl.BlockSpec((2, 1024), lambda i: (0, i))],
        out_specs=pl.BlockSpec((1, 1, 1024), lambda i: (i, 0, 0)),
        out_shape=jax.ShapeDtypeStruct((DEG_R // 1024, 1, 1024), jnp.float32),
    )(deg)


def _tc1_call(x, w1, dis3):
    def body(x_ref, w_ref, dis_ref, y_ref):
        dis = dis_ref[0, 0, :][:, None]
        xw = jnp.dot(x_ref[...], w_ref[...],
                     preferred_element_type=jnp.float32)
        y_ref[...] = xw * dis

    return pl.pallas_call(
        body,
        grid=(NB,),
        in_specs=[
            pl.BlockSpec((R, F), lambda i: (i, 0)),
            pl.BlockSpec((F, F), lambda i: (0, 0)),
            pl.BlockSpec((1, 1, R), lambda i: (i, 0, 0)),
        ],
        out_specs=pl.BlockSpec((R, F), lambda i: (i, 0)),
        out_shape=jax.ShapeDtypeStruct((N, F), jnp.float32),
    )(x, w1, dis3)


def _stats_call(agg, y, dis3, b):
    def body(agg_ref, y_ref, dis_ref, b_ref, pre_ref, st_ref):
        i = pl.program_id(0)
        dis = dis_ref[0, 0, :][:, None]
        pre = (agg_ref[...] + y_ref[...]) * dis + b_ref[...]
        pre_ref[...] = pre

        @pl.when(i == 0)
        def _():
            st_ref[...] = jnp.zeros_like(st_ref)

        st_ref[0:1, :] = st_ref[0:1, :] + jnp.sum(pre, 0, keepdims=True)
        st_ref[1:2, :] = st_ref[1:2, :] + jnp.sum(pre * pre, 0, keepdims=True)

    return pl.pallas_call(
        body,
        grid=(NB,),
        in_specs=[
            pl.BlockSpec((R, F), lambda i: (i, 0)),
            pl.BlockSpec((R, F), lambda i: (i, 0)),
            pl.BlockSpec((1, 1, R), lambda i: (i, 0, 0)),
            pl.BlockSpec((1, F), lambda i: (0, 0)),
        ],
        out_specs=[
            pl.BlockSpec((R, F), lambda i: (i, 0)),
            pl.BlockSpec((8, F), lambda i: (0, 0)),
        ],
        out_shape=[
            jax.ShapeDtypeStruct((N, F), jnp.float32),
            jax.ShapeDtypeStruct((8, F), jnp.float32),
        ],
    )(agg, y, dis3, b)


def _bn_relu(pre_ref, st_ref, g_ref, be_ref):
    mu = st_ref[0:1, :] / N
    var = st_ref[1:2, :] / N - mu * mu
    inv = lax.rsqrt(var + EPS)
    return jax.nn.relu((pre_ref[...] - mu) * inv * g_ref[...] + be_ref[...])


def _apply_call(pre, st, g, be, wn, dis3):
    def body(pre_ref, st_ref, g_ref, be_ref, w_ref, dis_ref, y_ref):
        h = _bn_relu(pre_ref, st_ref, g_ref, be_ref)
        dis = dis_ref[0, 0, :][:, None]
        y_ref[...] = jnp.dot(h, w_ref[...],
                             preferred_element_type=jnp.float32) * dis

    return pl.pallas_call(
        body,
        grid=(NB,),
        in_specs=[
            pl.BlockSpec((R, F), lambda i: (i, 0)),
            pl.BlockSpec((8, F), lambda i: (0, 0)),
            pl.BlockSpec((1, F), lambda i: (0, 0)),
            pl.BlockSpec((1, F), lambda i: (0, 0)),
            pl.BlockSpec((F, F), lambda i: (0, 0)),
            pl.BlockSpec((1, 1, R), lambda i: (i, 0, 0)),
        ],
        out_specs=pl.BlockSpec((R, F), lambda i: (i, 0)),
        out_shape=jax.ShapeDtypeStruct((N, F), jnp.float32),
    )(pre, st, g, be, wn, dis3)


def _pool_call(pre, st, g, be, batch2):
    def body(pre_ref, st_ref, g_ref, be_ref, b_ref, out_ref, acc, cnt):
        i = pl.program_id(0)
        h = _bn_relu(pre_ref, st_ref, g_ref, be_ref)

        @pl.when(i == 0)
        def _():
            acc[...] = jnp.zeros_like(acc)
            cnt[...] = jnp.zeros_like(cnt)

        bi = b_ref[0, 0, :]
        oh = (lax.broadcasted_iota(jnp.int32, (G, R), 0)
              == bi[None, :]).astype(jnp.float32)
        acc[...] = acc[...] + jnp.dot(oh, h,
                                      preferred_element_type=jnp.float32)
        cnt[...] = cnt[...] + jnp.sum(oh, 1, keepdims=True)
        out_ref[...] = acc[...] / jnp.maximum(cnt[...], 1.0)

    return pl.pallas_call(
        body,
        grid=(NB,),
        in_specs=[
            pl.BlockSpec((R, F), lambda i: (i, 0)),
            pl.BlockSpec((8, F), lambda i: (0, 0)),
            pl.BlockSpec((1, F), lambda i: (0, 0)),
            pl.BlockSpec((1, F), lambda i: (0, 0)),
            pl.BlockSpec((1, 1, R), lambda i: (i, 0, 0)),
        ],
        out_specs=pl.BlockSpec((G, F), lambda i: (0, 0)),
        out_shape=jax.ShapeDtypeStruct((G, F), jnp.float32),
        scratch_shapes=[
            pltpu.VMEM((G, F), jnp.float32),
            pltpu.VMEM((G, 1), jnp.float32),
        ],
    )(pre, st, g, be, batch2)


def kernel(x, edge_index, batch,
           W1, b1, g1, be1, W2, b2, g2, be2, W3, b3, g3, be3):
    f32 = jnp.float32
    src = edge_index[0]
    dst = edge_index[1]
    npad = E_PAD - E
    dst_pad = 50000 + (jnp.arange(npad, dtype=jnp.int32) % 1176)
    dst2 = jnp.concatenate([dst, dst_pad]).reshape(EROWS, 128)
    src2 = jnp.concatenate(
        [src, jnp.zeros((npad,), jnp.int32)]).reshape(EROWS, 128)
    cc = jnp.full((16,), NWIN, jnp.int32)

    w1p = jnp.pad(W1, ((0, 0), (0, F - W1.shape[1]))).astype(f32)
    w2p = jnp.pad(W2, ((0, F - W2.shape[0]), (0, 0))).astype(f32)
    pad1 = lambda v: jnp.pad(v, (0, F - v.shape[0])).reshape(1, F).astype(f32)
    b1p, g1p, be1p = pad1(b1), pad1(g1), pad1(be1)
    b2p, g2p, be2p = pad1(b2), pad1(g2), pad1(be2)
    b3p, g3p, be3p = pad1(b3), pad1(g3), pad1(be3)
    batch2 = batch.reshape(NB, 1, R)

    deg = _deg_call(dst2)
    bsrc, bdst, bcnt = _bin_call(src2, dst2, cc)
    dis3 = _dis_call(deg).reshape(DEG_R)[:N].reshape(NB, 1, R)

    y1 = _tc1_call(x, w1p, dis3)
    agg1 = _agg_call(y1, bsrc, bdst, bcnt)
    pre1, st1 = _stats_call(agg1, y1, dis3, b1p)

    y2 = _apply_call(pre1, st1, g1p, be1p, w2p, dis3)
    agg2 = _agg_call(y2, bsrc, bdst, bcnt)
    pre2, st2 = _stats_call(agg2, y2, dis3, b2p)

    y3 = _apply_call(pre2, st2, g2p, be2p, W3.astype(f32), dis3)
    agg3 = _agg_call(y3, bsrc, bdst, bcnt)
    pre3, st3 = _stats_call(agg3, y3, dis3, b3p)

    return _pool_call(pre3, st3, g3p, be3p, batch2)
